# baseline (speedup 1.0000x reference)
import functools

import jax
import jax.numpy as jnp
from jax import lax
from jax.experimental import pallas as pl
from jax.experimental.pallas import tpu as pltpu
from jax.experimental.pallas import tpu_sc as plsc

NC = 2
NS = 16
NW = NC * NS
L = 16
CN = 1024
CAP = 1536
SCAP = 1792


def kernel(attr_map, weight_u, weight_v, u_low, v_low, u_high, v_high):
    B, H, W, C = attr_map.shape
    N = u_low.shape[0]
    BN = B * N
    NV = H // NW
    Npad = -(-N // CN) * CN
    VBIG = jnp.int32(1 << 20)

    tableT = attr_map.transpose(0, 1, 3, 2)
    wu = weight_u.reshape(N)
    wv = weight_v.reshape(N)
    if Npad != N:
        pad = Npad - N
        zi = jnp.zeros((pad,), jnp.int32)
        zf = jnp.zeros((pad,), jnp.float32)
        ul = jnp.concatenate([u_low, zi])
        vl = jnp.concatenate([v_low, zi])
        uh = jnp.concatenate([u_high, zi])
        vh = jnp.concatenate([v_high, jnp.full((pad,), VBIG, jnp.int32)])
        wu = jnp.concatenate([wu, zf])
        wv = jnp.concatenate([wv, zf])
    else:
        ul, vl, uh, vh = u_low, v_low, u_high, v_high

    def body(tT_h, ul_h, vl_h, uh_h, vh_h, wu_h, wv_h, outp_h,
             vh_s, vl_s, ul_s, uh_s, wu_s, wv_s,
             pu_u, n_u, s1_u, s2_u, s3_u,
             pu_s, n_s, s1_s, s2_s, s3_s,
             ring, o_v, sidx, offs, ssem, gsem, osem):
        w = lax.axis_index("s") * NC + lax.axis_index("c")
        vlo = w * NV
        iota = lax.iota(jnp.int32, L)
        civs = [iota + cc * L for cc in range(C // L)]
        SENT = jnp.full((L,), 31 << 21, jnp.int32)

        def fill_u(k, _):
            pu_u[pl.ds(k * L, L)] = SENT
            return 0
        lax.fori_loop(0, (CAP + L) // L, fill_u, 0)

        dummy_pu = jnp.full((L,), 1 << 20, jnp.int32)
        dummy_n = jnp.full((L,), BN, jnp.int32)
        zf16 = jnp.zeros((L,), jnp.float32)

        def fill_s(k, _):
            sl = pl.ds(k * L, L)
            pu_s[sl] = dummy_pu
            n_s[sl] = dummy_n
            s1_s[sl] = zf16
            s2_s[sl] = zf16
            s3_s[sl] = zf16
            return 0
        lax.fori_loop(0, (SCAP + L) // L, fill_s, 0)

        def scan_chunk(ci, pos):
            cb = ci * CN
            cps = [pltpu.async_copy(vh_h.at[pl.ds(cb, CN)], vh_s, ssem),
                   pltpu.async_copy(vl_h.at[pl.ds(cb, CN)], vl_s, ssem),
                   pltpu.async_copy(ul_h.at[pl.ds(cb, CN)], ul_s, ssem),
                   pltpu.async_copy(uh_h.at[pl.ds(cb, CN)], uh_s, ssem),
                   pltpu.async_copy(wu_h.at[pl.ds(cb, CN)], wu_s, ssem),
                   pltpu.async_copy(wv_h.at[pl.ds(cb, CN)], wv_s, ssem)]
            for cp in cps:
                cp.wait()

            def vec_iter(k, pos):
                sl = pl.ds(k * L, L)
                vhv = vh_s[sl]
                m = (vhv >= vlo) & (vhv < vlo + NV)
                cnt = plsc.all_reduce_population_count(m)[0]
                vlv = vl_s[sl]
                ulv = ul_s[sl]
                uhv = uh_s[sl]
                d = (vhv - vlv) & 1
                rel = (vhv - vlo) & 15
                pu = ulv | (uhv << 10) | (d << 20) | (rel << 21)
                nv_ = cb + k * L + iota
                wuv = wu_s[sl]
                wvv = wv_s[sl]
                t1 = 1.0 - wuv
                s2 = t1 * wvv
                s3 = t1 - s2
                p0 = jnp.minimum(pos, CAP)
                plsc.store_compressed(pu_u.at[pl.ds(p0, L)], pu, mask=m)
                plsc.store_compressed(n_u.at[pl.ds(p0, L)], nv_, mask=m)
                plsc.store_compressed(s1_u.at[pl.ds(p0, L)], wuv, mask=m)
                plsc.store_compressed(s2_u.at[pl.ds(p0, L)], s2, mask=m)
                plsc.store_compressed(s3_u.at[pl.ds(p0, L)], s3, mask=m)
                return jnp.minimum(pos + cnt, CAP)
            return lax.fori_loop(0, CN // L, vec_iter, pos)
        m_total = lax.fori_loop(0, Npad // CN, scan_chunk, jnp.int32(0))
        mvec = (m_total + L - 1) // L

        spos = jnp.int32(0)
        for v16 in range(NV):
            offs[v16] = spos

            def bin_iter(k, spos, v16=v16):
                sl = pl.ds(k * L, L)
                pv = pu_u[sl]
                m = ((pv >> 21) & 31) == v16
                cnt = plsc.all_reduce_population_count(m)[0]
                sp = pl.ds(spos, L)
                plsc.store_compressed(pu_s.at[sp], pv, mask=m)
                plsc.store_compressed(n_s.at[sp], n_u[sl], mask=m)
                plsc.store_compressed(s1_s.at[sp], s1_u[sl], mask=m)
                plsc.store_compressed(s2_s.at[sp], s2_u[sl], mask=m)
                plsc.store_compressed(s3_s.at[sp], s3_u[sl], mask=m)
                return spos + cnt
            spos = lax.fori_loop(0, mvec, bin_iter, spos)
            gp = pl.ds(spos, L)
            pu_s[gp] = dummy_pu
            n_s[gp] = dummy_n
            s1_s[gp] = zf16
            s2_s[gp] = zf16
            s3_s[gp] = zf16
            spos = ((spos + L - 1) // L) * L
        offs[NV] = spos

        def batch_body(b, _):
            pltpu.async_copy(
                tT_h.at[b, jnp.maximum(vlo - 1, 0)], ring.at[1], gsem).wait()

            def v_body(v16, _):
                v = vlo + v16
                slot = v16 & 1
                pltpu.async_copy(tT_h.at[b, v], ring.at[slot], gsem).wait()
                off0 = offs[v16]
                ng = (offs[v16 + 1] - off0) // L
                curv = jnp.full((L,), slot, jnp.int32)

                def g_body(g, _):
                    base = off0 + g * L
                    gs = g & 1
                    pv = pu_s[pl.ds(base, L)]
                    nv_ = n_s[pl.ds(base, L)]
                    s1v = s1_s[pl.ds(base, L)]
                    s2v = s2_s[pl.ds(base, L)]
                    s3v = s3_s[pl.ds(base, L)]
                    sidx[gs] = jnp.minimum(nv_ + b * N, BN)
                    for t in range(L):
                        p = pv[t]
                        u0 = p & 1023
                        u1 = (p >> 10) & 1023
                        d = (p >> 20) & 1
                        s00 = slot ^ d
                        a1 = s1v[t]
                        a2 = s2v[t]
                        a3 = s3v[t]
                        u0v = jnp.full((L,), u0, jnp.int32)
                        u1v = jnp.full((L,), u1, jnp.int32)
                        s00v = jnp.full((L,), s00, jnp.int32)
                        for cc in range(C // L):
                            civ = civs[cc]
                            g11 = plsc.load_gather(ring, [curv, civ, u1v])
                            g01 = plsc.load_gather(ring, [curv, civ, u0v])
                            g00 = plsc.load_gather(ring, [s00v, civ, u0v])
                            o_v[gs, t, pl.ds(cc * L, L)] = (
                                a1 * g11 + a2 * g01 + a3 * g00)
                    pltpu.async_copy(
                        o_v.at[gs], outp_h.at[sidx.at[gs]], osem).wait()
                    return 0
                lax.fori_loop(0, ng, g_body, 0)
                return 0
            lax.fori_loop(0, NV, v_body, 0)
            return 0
        lax.fori_loop(0, B, batch_body, 0)

    mesh = plsc.VectorSubcoreMesh(core_axis_name="c", subcore_axis_name="s",
                                  num_cores=NC, num_subcores=NS)
    f = pl.kernel(
        body,
        out_type=jax.ShapeDtypeStruct((BN + 8, 128), jnp.float32),
        mesh=mesh,
        compiler_params=pltpu.CompilerParams(needs_layout_passes=False),
        scratch_types=[
            pltpu.VMEM((CN,), jnp.int32),
            pltpu.VMEM((CN,), jnp.int32),
            pltpu.VMEM((CN,), jnp.int32),
            pltpu.VMEM((CN,), jnp.int32),
            pltpu.VMEM((CN,), jnp.float32),
            pltpu.VMEM((CN,), jnp.float32),
            pltpu.VMEM((CAP + L,), jnp.int32),
            pltpu.VMEM((CAP + L,), jnp.int32),
            pltpu.VMEM((CAP + L,), jnp.float32),
            pltpu.VMEM((CAP + L,), jnp.float32),
            pltpu.VMEM((CAP + L,), jnp.float32),
            pltpu.VMEM((SCAP + L,), jnp.int32),
            pltpu.VMEM((SCAP + L,), jnp.int32),
            pltpu.VMEM((SCAP + L,), jnp.float32),
            pltpu.VMEM((SCAP + L,), jnp.float32),
            pltpu.VMEM((SCAP + L,), jnp.float32),
            pltpu.VMEM((2, C, W), jnp.float32),
            pltpu.VMEM((2, L, 128), jnp.float32),
            pltpu.VMEM((2, L), jnp.int32),
            pltpu.SMEM((NV + 1,), jnp.int32),
            pltpu.SemaphoreType.DMA,
            pltpu.SemaphoreType.DMA,
            pltpu.SemaphoreType.DMA,
        ],
    )
    outp = f(tableT, ul, vl, uh, vh, wu, wv)
    return outp[:BN, :C].reshape(B, N, C)
